# Initial kernel scaffold; baseline (speedup 1.0000x reference)
import jax
import jax.numpy as jnp
from jax.experimental import pallas as pl


def kernel(X, tables):
    raise NotImplementedError("write your pallas kernel here")


# speedup vs baseline: 2.7739x; 2.7739x over previous
import functools

import jax
import jax.numpy as jnp
from jax import lax
from jax.experimental import pallas as pl
from jax.experimental.pallas import tpu as pltpu
from jax.experimental.pallas import tpu_sc as plsc

B = 1024
C = 26
S = 50
VOCAB = 100000
EMB = 32

N = B * C * S
NW = 32
PER_W = N // NW
K = 1600
NCHUNK = PER_W // K
LANES = 16

_mesh = plsc.VectorSubcoreMesh(core_axis_name="c", subcore_axis_name="s")


@functools.partial(
    pl.kernel,
    mesh=_mesh,
    out_type=jax.ShapeDtypeStruct((N, EMB), jnp.float32),
    scratch_types=[
        pltpu.VMEM((K,), jnp.int32),
        pltpu.VMEM((K,), jnp.int32),
        pltpu.VMEM((K, EMB), jnp.float32),
        pltpu.SemaphoreType.DMA,
    ],
    compiler_params=pltpu.CompilerParams(use_tc_tiling_on_sc=False),
)
def _gather_kernel(x_hbm, tab_hbm, offs_hbm, out_hbm, idx_v, offs_v, rows_v, sem):
    wid = lax.axis_index("s") * 2 + lax.axis_index("c")
    base = wid * PER_W

    def chunk_body(ci, carry):
        cbase = base + ci * K
        pltpu.sync_copy(x_hbm.at[pl.ds(cbase, K)], idx_v)
        pltpu.sync_copy(offs_hbm.at[pl.ds(cbase, K)], offs_v)

        def off_body(j, carry2):
            sl = pl.ds(j * LANES, LANES)
            idx_v[sl] = idx_v[sl] + offs_v[sl]
            return carry2

        lax.fori_loop(0, K // LANES, off_body, 0)

        pltpu.async_copy(tab_hbm.at[idx_v], rows_v, sem).wait()
        pltpu.sync_copy(rows_v, out_hbm.at[pl.ds(cbase, K)])
        return carry

    lax.fori_loop(0, NCHUNK, chunk_body, 0)


def kernel(X, tables):
    x_flat = X.reshape(N)
    tab_flat = tables.reshape(C * VOCAB, EMB)
    offs = jnp.tile(jnp.repeat(jnp.arange(C, dtype=jnp.int32) * VOCAB, S), B)
    y = _gather_kernel(x_flat, tab_flat, offs)
    return y.reshape(B, C, S, EMB)
